# baseline (speedup 1.0000x reference)
import functools

import jax
import jax.numpy as jnp
from jax import lax
from jax.experimental import pallas as pl
from jax.experimental.pallas import tpu as pltpu
from jax.experimental.pallas import tpu_sc as plsc

NH = 16
WIN = 32
DIAG = 2 * WIN - 1
TROWS = DIAG * DIAG
UCOLS = DIAG * WIN
CHUNK = 32 * DIAG

_mesh = plsc.VectorSubcoreMesh(core_axis_name="c", subcore_axis_name="s")


@functools.partial(
    pl.kernel,
    mesh=_mesh,
    compiler_params=pltpu.CompilerParams(
        use_tc_tiling_on_sc=False, needs_layout_passes=False
    ),
    out_type=jax.ShapeDtypeStruct((NH, WIN * WIN, WIN * WIN), jnp.float32),
    scratch_types=[
        pltpu.VMEM((TROWS * NH,), jnp.float32),
        pltpu.VMEM((16, UCOLS), jnp.float32),
        pltpu.SemaphoreType.DMA,
        pltpu.SemaphoreType.DMA,
        pltpu.SemaphoreType.DMA,
        pltpu.SemaphoreType.DMA,
    ],
)
def _sc_bias(table_hbm, out_hbm, tbl_v, u_v, tsem_a1, tsem_a2, tsem_b, osem):
    cid = lax.axis_index("c")
    sid = lax.axis_index("s")
    wid = sid * 2 + cid
    h = wid // 2
    iw_base = (wid % 2) * 16

    def stage(lo_row, hi_row, sem):
        return pltpu.make_async_copy(
            table_hbm.at[pl.ds(lo_row * NH, (hi_row - lo_row) * NH)],
            tbl_v.at[pl.ds(lo_row * NH, (hi_row - lo_row) * NH)],
            sem)

    copy_a1 = stage(0, 8 * DIAG, tsem_a1)
    copy_a2 = stage(8 * DIAG, CHUNK, tsem_a2)
    copy_b = stage(CHUNK, TROWS, tsem_b)
    copy_a1.start()
    copy_a2.start()
    copy_b.start()

    lane16 = lax.iota(jnp.int32, 16) * NH

    def gather_cc(cc):
        ubase = (62 - cc) * WIN
        rowbase = cc * DIAG * NH + h
        for r in range(16):
            iw = iw_base + r
            for jwb in (0, 16):
                idx = jnp.full((16,), 0, jnp.int32) + (
                    rowbase + (iw - jwb + 31) * NH) - lane16
                u_v[r, pl.ds(ubase + jwb, 16)] = plsc.load_gather(tbl_v, [idx])

    def gather_range(lo, hi):
        def body(cc, carry):
            gather_cc(cc)
            return carry
        lax.fori_loop(lo, hi, body, 0, unroll=2)

    def fire_window(cc):
        ih = cc - (WIN - 1)
        pltpu.make_async_copy(
            u_v.at[:, pl.ds((62 - cc) * WIN, WIN * WIN)],
            out_hbm.at[h, pl.ds(ih * WIN + iw_base, 16), :],
            osem,
        ).start()

    copy_a1.wait()
    gather_range(0, 8)
    copy_a2.wait()
    gather_range(8, WIN)
    fire_window(WIN - 1)

    copy_b.wait()

    def body2(cc, carry):
        gather_cc(cc)
        fire_window(cc)
        return carry

    lax.fori_loop(WIN, DIAG, body2, 0)

    for _ in range(WIN):
        pltpu.make_async_copy(
            u_v.at[:, pl.ds(0, WIN * WIN)],
            out_hbm.at[0, pl.ds(iw_base, 16), :],
            osem,
        ).wait()


def kernel(attn, relative_position_bias_table, relative_position_index):
    del attn, relative_position_index
    bias = _sc_bias(relative_position_bias_table.reshape(-1))
    return bias[None, :, :, :, None]


# speedup vs baseline: 1.0103x; 1.0103x over previous
import functools

import jax
import jax.numpy as jnp
from jax import lax
from jax.experimental import pallas as pl
from jax.experimental.pallas import tpu as pltpu
from jax.experimental.pallas import tpu_sc as plsc

NH = 16
WIN = 32
DIAG = 2 * WIN - 1
TROWS = DIAG * DIAG
UCOLS = DIAG * WIN
CHUNK = 32 * DIAG

_mesh = plsc.VectorSubcoreMesh(core_axis_name="c", subcore_axis_name="s")


@functools.partial(
    pl.kernel,
    mesh=_mesh,
    compiler_params=pltpu.CompilerParams(
        use_tc_tiling_on_sc=False, needs_layout_passes=False
    ),
    out_type=jax.ShapeDtypeStruct((NH, WIN * WIN, WIN * WIN), jnp.float32),
    scratch_types=[
        pltpu.VMEM((TROWS * NH,), jnp.float32),
        pltpu.VMEM((16, UCOLS), jnp.float32),
        pltpu.SemaphoreType.DMA,
        pltpu.SemaphoreType.DMA,
        pltpu.SemaphoreType.DMA,
        pltpu.SemaphoreType.DMA,
    ],
)
def _sc_bias(table_hbm, out_hbm, tbl_v, u_v, tsem_a1, tsem_a2, tsem_b, osem):
    cid = lax.axis_index("c")
    sid = lax.axis_index("s")
    wid = sid * 2 + cid
    h = wid // 2
    iw_base = (wid % 2) * 16

    def stage(lo_row, hi_row, sem):
        return pltpu.make_async_copy(
            table_hbm.at[pl.ds(lo_row * NH, (hi_row - lo_row) * NH)],
            tbl_v.at[pl.ds(lo_row * NH, (hi_row - lo_row) * NH)],
            sem)

    copy_a1 = stage(0, 8 * DIAG, tsem_a1)
    copy_a2 = stage(8 * DIAG, CHUNK, tsem_a2)
    copy_b = stage(CHUNK, TROWS, tsem_b)
    copy_a1.start()
    copy_a2.start()
    copy_b.start()

    lane16 = lax.iota(jnp.int32, 16) * NH

    def gather_cc(cc):
        ubase = (62 - cc) * WIN
        rowbase = cc * DIAG * NH + h
        for r in range(16):
            iw = iw_base + r
            for jwb in (0, 16):
                idx = jnp.full((16,), 0, jnp.int32) + (
                    rowbase + (iw - jwb + 31) * NH) - lane16
                u_v[r, pl.ds(ubase + jwb, 16)] = plsc.load_gather(tbl_v, [idx])

    def gather_range(lo, hi):
        def body(cc, carry):
            gather_cc(cc)
            return carry
        lax.fori_loop(lo, hi, body, 0)

    def fire_window(cc):
        ih = cc - (WIN - 1)
        pltpu.make_async_copy(
            u_v.at[:, pl.ds((62 - cc) * WIN, WIN * WIN)],
            out_hbm.at[h, pl.ds(ih * WIN + iw_base, 16), :],
            osem,
        ).start()

    copy_a1.wait()
    gather_range(0, 8)
    copy_a2.wait()
    gather_range(8, WIN)
    fire_window(WIN - 1)

    copy_b.wait()

    def body2(cc, carry):
        gather_cc(cc)
        fire_window(cc)
        return carry

    lax.fori_loop(WIN, DIAG, body2, 0)

    for _ in range(WIN):
        pltpu.make_async_copy(
            u_v.at[:, pl.ds(0, WIN * WIN)],
            out_hbm.at[0, pl.ds(iw_base, 16), :],
            osem,
        ).wait()


def kernel(attn, relative_position_bias_table, relative_position_index):
    del attn, relative_position_index
    bias = _sc_bias(relative_position_bias_table.reshape(-1))
    return bias[None, :, :, :, None]
